# Initial kernel scaffold; baseline (speedup 1.0000x reference)
import jax
import jax.numpy as jnp
from jax.experimental import pallas as pl


def kernel(segments, semantic_embeds, categories, W, b, E0, E1, E2):
    raise NotImplementedError("write your pallas kernel here")


# speedup vs baseline: 3.1352x; 3.1352x over previous
import functools
import math

import jax
import jax.numpy as jnp
import numpy as np
from jax import lax
from jax.experimental import pallas as pl
from jax.experimental.pallas import tpu as pltpu
from jax.experimental.pallas import tpu_sc as plsc

B, S, EMBED_LEN, HIDDEN = 4096, 50, 128, 64
MAX_LEN = 256

NC, NS, L = 2, 16, 16
NW = NC * NS
N_TOK = B * S
TOK_PER_W = N_TOK // NW
CHUNK = 128
N_CHUNK = TOK_PER_W // CHUNK
VECS_PER_ROW = HIDDEN // L


def _pe_np(d_model=HIDDEN, max_len=MAX_LEN):
    position = np.arange(0, max_len, dtype=np.float32)[:, None]
    div_term = np.exp(
        np.arange(0, d_model, 2, dtype=np.float32) * -(math.log(10000.0) / d_model))
    pe = np.zeros((max_len, d_model), dtype=np.float32)
    odd_len = d_model - div_term.shape[-1]
    pe[:, 0::2] = np.sin(position * div_term)
    pe[:, 1::2] = np.cos(position * div_term[:odd_len])
    return pe


_PE = _pe_np()[:S]


def _sc_gather_sum(c0, c1, c2, e0, e1, e2):
    mesh = plsc.VectorSubcoreMesh(
        core_axis_name="c", subcore_axis_name="s",
        num_cores=NC, num_subcores=NS)

    @functools.partial(
        pl.kernel,
        out_type=jax.ShapeDtypeStruct((N_TOK, HIDDEN), jnp.float32),
        mesh=mesh,
        compiler_params=pltpu.CompilerParams(use_tc_tiling_on_sc=False),
        scratch_types=[
            pltpu.VMEM((CHUNK,), jnp.int32),
            pltpu.VMEM((CHUNK,), jnp.int32),
            pltpu.VMEM((CHUNK,), jnp.int32),
            pltpu.VMEM((CHUNK, HIDDEN), jnp.float32),
            pltpu.VMEM((CHUNK, HIDDEN), jnp.float32),
            pltpu.VMEM((CHUNK, HIDDEN), jnp.float32),
            pltpu.SemaphoreType.DMA,
            pltpu.SemaphoreType.DMA,
        ],
    )
    def gather_sum(c0_hbm, c1_hbm, c2_hbm, e0_hbm, e1_hbm, e2_hbm, g_hbm,
                   i0, i1, i2, r0, r1, r2, gsem, wsem):
        wid = lax.axis_index("s") * NC + lax.axis_index("c")
        base = wid * TOK_PER_W

        def step(i, carry):
            tok = base + i * CHUNK
            pltpu.sync_copy(c0_hbm.at[pl.ds(tok, CHUNK)], i0)
            pltpu.sync_copy(c1_hbm.at[pl.ds(tok, CHUNK)], i1)
            pltpu.sync_copy(c2_hbm.at[pl.ds(tok, CHUNK)], i2)
            d0 = pltpu.async_copy(e0_hbm.at[i0], r0, gsem)
            d1 = pltpu.async_copy(e1_hbm.at[i1], r1, gsem)
            d2 = pltpu.async_copy(e2_hbm.at[i2], r2, gsem)
            d0.wait()
            d1.wait()
            d2.wait()

            def acc(j, c):
                row = j // VECS_PER_ROW
                col = (j % VECS_PER_ROW) * L
                r0[row, pl.ds(col, L)] = (
                    r0[row, pl.ds(col, L)]
                    + r1[row, pl.ds(col, L)]
                    + r2[row, pl.ds(col, L)])
                return c

            lax.fori_loop(0, CHUNK * VECS_PER_ROW, acc, 0)
            pltpu.async_copy(r0, g_hbm.at[pl.ds(tok, CHUNK)], wsem).wait()
            return carry

        lax.fori_loop(0, N_CHUNK, step, 0)

    return gather_sum(c0, c1, c2, e0, e1, e2)


BB = 128


def _tc_body(sem_ref, w_ref, pb_ref, g_ref, o_ref):
    x = sem_ref[...].reshape(BB * S, EMBED_LEN)
    y = jnp.dot(x, w_ref[...], preferred_element_type=jnp.float32)
    o_ref[...] = y.reshape(BB, S, HIDDEN) + pb_ref[...][None] + g_ref[...]


def kernel(segments, semantic_embeds, categories, W, b, E0, E1, E2):
    del segments
    cflat = categories.reshape(N_TOK, 3)
    c0 = cflat[:, 0]
    c1 = cflat[:, 1]
    c2 = cflat[:, 2]

    g = _sc_gather_sum(c0, c1, c2, E0, E1, E2).reshape(B, S, HIDDEN)
    pe_b = jnp.asarray(_PE) + b[None, :]

    out = pl.pallas_call(
        _tc_body,
        grid=(B // BB,),
        in_specs=[
            pl.BlockSpec((BB, S, EMBED_LEN), lambda i: (i, 0, 0)),
            pl.BlockSpec((EMBED_LEN, HIDDEN), lambda i: (0, 0)),
            pl.BlockSpec((S, HIDDEN), lambda i: (0, 0)),
            pl.BlockSpec((BB, S, HIDDEN), lambda i: (i, 0, 0)),
        ],
        out_specs=pl.BlockSpec((BB, S, HIDDEN), lambda i: (i, 0, 0)),
        out_shape=jax.ShapeDtypeStruct((B, S, HIDDEN), jnp.float32),
    )(semantic_embeds, W, pe_b, g)
    return out


# speedup vs baseline: 4.4539x; 1.4206x over previous
import functools
import math

import jax
import jax.numpy as jnp
import numpy as np
from jax import lax
from jax.experimental import pallas as pl
from jax.experimental.pallas import tpu as pltpu
from jax.experimental.pallas import tpu_sc as plsc

B, S, EMBED_LEN, HIDDEN = 4096, 50, 128, 64
MAX_LEN = 256

NC, NS, L = 2, 16, 16
NW = NC * NS
N_TOK = B * S
N2 = N_TOK // 2
TOK_PER_W = N_TOK // NW
CHUNK = 128
N_CHUNK = TOK_PER_W // CHUNK
PAIRS = CHUNK // 2
ROWS_PER_W = TOK_PER_W // 2


def _pe_np(d_model=HIDDEN, max_len=MAX_LEN):
    position = np.arange(0, max_len, dtype=np.float32)[:, None]
    div_term = np.exp(
        np.arange(0, d_model, 2, dtype=np.float32) * -(math.log(10000.0) / d_model))
    pe = np.zeros((max_len, d_model), dtype=np.float32)
    odd_len = d_model - div_term.shape[-1]
    pe[:, 0::2] = np.sin(position * div_term)
    pe[:, 1::2] = np.cos(position * div_term[:odd_len])
    return pe


_PE = _pe_np()[:S]


def _sc_gather_sum(ct, e0, e1, e2):
    mesh = plsc.VectorSubcoreMesh(
        core_axis_name="c", subcore_axis_name="s",
        num_cores=NC, num_subcores=NS)

    @functools.partial(
        pl.kernel,
        out_type=jax.ShapeDtypeStruct((N2, 2 * HIDDEN), jnp.float32),
        mesh=mesh,
        compiler_params=pltpu.CompilerParams(use_tc_tiling_on_sc=False),
        scratch_types=[
            pltpu.VMEM((TOK_PER_W,), jnp.int32),
            pltpu.VMEM((TOK_PER_W,), jnp.int32),
            pltpu.VMEM((TOK_PER_W,), jnp.int32),
            pltpu.VMEM((2, CHUNK, HIDDEN), jnp.float32),
            pltpu.VMEM((2, CHUNK, HIDDEN), jnp.float32),
            pltpu.VMEM((2, CHUNK, HIDDEN), jnp.float32),
            pltpu.VMEM((2, PAIRS, 2 * HIDDEN), jnp.float32),
            pltpu.SemaphoreType.DMA,
            pltpu.SemaphoreType.DMA,
            pltpu.SemaphoreType.DMA,
            pltpu.SemaphoreType.DMA,
        ],
    )
    def gather_sum(ct_hbm, e0_hbm, e1_hbm, e2_hbm, g_hbm,
                   i0, i1, i2, r0, r1, r2, acc, gsem0, gsem1, wsem0, wsem1):
        wid = lax.axis_index("s") * NC + lax.axis_index("c")
        tbase = wid * TOK_PER_W
        rbase = wid * ROWS_PER_W
        pltpu.sync_copy(ct_hbm.at[0, pl.ds(tbase, TOK_PER_W)], i0)
        pltpu.sync_copy(ct_hbm.at[1, pl.ds(tbase, TOK_PER_W)], i1)
        pltpu.sync_copy(ct_hbm.at[2, pl.ds(tbase, TOK_PER_W)], i2)

        gsems = (gsem0, gsem1)
        wsems = (wsem0, wsem1)

        def issue(j, p):
            off = j * CHUNK
            pltpu.async_copy(e0_hbm.at[i0.at[pl.ds(off, CHUNK)]], r0.at[p], gsems[p])
            pltpu.async_copy(e1_hbm.at[i1.at[pl.ds(off, CHUNK)]], r1.at[p], gsems[p])
            pltpu.async_copy(e2_hbm.at[i2.at[pl.ds(off, CHUNK)]], r2.at[p], gsems[p])

        def wait_gather(j, p):
            off = j * CHUNK
            pltpu.make_async_copy(
                e0_hbm.at[i0.at[pl.ds(off, CHUNK)]], r0.at[p], gsems[p]).wait()
            pltpu.make_async_copy(
                e1_hbm.at[i1.at[pl.ds(off, CHUNK)]], r1.at[p], gsems[p]).wait()
            pltpu.make_async_copy(
                e2_hbm.at[i2.at[pl.ds(off, CHUNK)]], r2.at[p], gsems[p]).wait()

        def compute(p):
            def body(q, carry):
                for half in range(2):
                    tok = 2 * q + half
                    for k in range(HIDDEN // L):
                        col = half * HIDDEN + k * L
                        acc[p, q, pl.ds(col, L)] = (
                            r0[p, tok, pl.ds(k * L, L)]
                            + r1[p, tok, pl.ds(k * L, L)]
                            + r2[p, tok, pl.ds(k * L, L)])
                return carry
            lax.fori_loop(0, PAIRS, body, 0)

        def issue_write(j, p):
            pltpu.async_copy(
                acc.at[p], g_hbm.at[pl.ds(rbase + j * PAIRS, PAIRS)], wsems[p])

        def wait_write(j, p):
            pltpu.make_async_copy(
                acc.at[p], g_hbm.at[pl.ds(rbase + j * PAIRS, PAIRS)], wsems[p]).wait()

        issue(0, 0)

        def step(t, carry):
            issue(2 * t + 1, 1)
            wait_gather(2 * t, 0)

            @pl.when(t > 0)
            def _():
                wait_write(2 * t - 2, 0)

            compute(0)
            issue_write(2 * t, 0)

            @pl.when(t < N_CHUNK // 2 - 1)
            def _():
                issue(2 * t + 2, 0)

            wait_gather(2 * t + 1, 1)

            @pl.when(t > 0)
            def _():
                wait_write(2 * t - 1, 1)

            compute(1)
            issue_write(2 * t + 1, 1)
            return carry

        lax.fori_loop(0, N_CHUNK // 2, step, 0)
        wait_write(N_CHUNK - 2, 0)
        wait_write(N_CHUNK - 1, 1)

    return gather_sum(ct, e0, e1, e2)


TB2 = 3200


def _tc_body(x_ref, w2_ref, pb_ref, g_ref, o_ref):
    o_ref[...] = (
        jnp.dot(x_ref[...], w2_ref[...], preferred_element_type=jnp.float32)
        + pb_ref[...] + g_ref[...])


def kernel(segments, semantic_embeds, categories, W, b, E0, E1, E2):
    del segments
    ct = categories.transpose(2, 0, 1).reshape(3, N_TOK)
    g2 = _sc_gather_sum(ct, E0, E1, E2)

    sem2 = semantic_embeds.reshape(N2, 2 * EMBED_LEN)
    w2 = jnp.zeros((2 * EMBED_LEN, 2 * HIDDEN), dtype=jnp.float32)
    w2 = w2.at[:EMBED_LEN, :HIDDEN].set(W).at[EMBED_LEN:, HIDDEN:].set(W)
    pe_b = jnp.asarray(_PE) + b[None, :]
    pb2 = jnp.tile(pe_b.reshape(S // 2, 2 * HIDDEN), (TB2 // (S // 2), 1))

    out2 = pl.pallas_call(
        _tc_body,
        grid=(N2 // TB2,),
        in_specs=[
            pl.BlockSpec((TB2, 2 * EMBED_LEN), lambda i: (i, 0)),
            pl.BlockSpec((2 * EMBED_LEN, 2 * HIDDEN), lambda i: (0, 0)),
            pl.BlockSpec((TB2, 2 * HIDDEN), lambda i: (0, 0)),
            pl.BlockSpec((TB2, 2 * HIDDEN), lambda i: (i, 0)),
        ],
        out_specs=pl.BlockSpec((TB2, 2 * HIDDEN), lambda i: (i, 0)),
        out_shape=jax.ShapeDtypeStruct((N2, 2 * HIDDEN), jnp.float32),
    )(sem2, w2, pb2, g2)
    return out2.reshape(B, S, HIDDEN)


# speedup vs baseline: 6.6817x; 1.5002x over previous
import functools
import math

import jax
import jax.numpy as jnp
import numpy as np
from jax import lax
from jax.experimental import pallas as pl
from jax.experimental.pallas import tpu as pltpu
from jax.experimental.pallas import tpu_sc as plsc

B, S, EMBED_LEN, HIDDEN = 4096, 50, 128, 64
MAX_LEN = 256

NC, NS, L = 2, 16, 16
NW = NC * NS
N_TOK = B * S
N2 = N_TOK // 2
TOK_PER_W = N_TOK // NW
CHUNK = 128
N_CHUNK = TOK_PER_W // CHUNK
VPT = HIDDEN // L


def _pe_np(d_model=HIDDEN, max_len=MAX_LEN):
    position = np.arange(0, max_len, dtype=np.float32)[:, None]
    div_term = np.exp(
        np.arange(0, d_model, 2, dtype=np.float32) * -(math.log(10000.0) / d_model))
    pe = np.zeros((max_len, d_model), dtype=np.float32)
    odd_len = d_model - div_term.shape[-1]
    pe[:, 0::2] = np.sin(position * div_term)
    pe[:, 1::2] = np.cos(position * div_term[:odd_len])
    return pe


_PE = _pe_np()[:S]


def _sc_gather_scatter_sum(ct, oidx2, e0, e1, e2):
    mesh = plsc.VectorSubcoreMesh(
        core_axis_name="c", subcore_axis_name="s",
        num_cores=NC, num_subcores=NS)

    @functools.partial(
        pl.kernel,
        out_type=jax.ShapeDtypeStruct((N_TOK, HIDDEN), jnp.float32),
        mesh=mesh,
        compiler_params=pltpu.CompilerParams(use_tc_tiling_on_sc=False),
        scratch_types=[
            pltpu.VMEM((TOK_PER_W,), jnp.int32),
            pltpu.VMEM((TOK_PER_W,), jnp.int32),
            pltpu.VMEM((TOK_PER_W,), jnp.int32),
            pltpu.VMEM((N_CHUNK, CHUNK), jnp.int32),
            pltpu.VMEM((2, CHUNK, HIDDEN), jnp.float32),
            pltpu.VMEM((2, CHUNK, HIDDEN), jnp.float32),
            pltpu.VMEM((2, CHUNK, HIDDEN), jnp.float32),
            pltpu.VMEM((2, CHUNK, HIDDEN), jnp.float32),
            pltpu.SemaphoreType.DMA,
            pltpu.SemaphoreType.DMA,
            pltpu.SemaphoreType.DMA,
            pltpu.SemaphoreType.DMA,
        ],
    )
    def gather_sum(ct_hbm, oidx_hbm, e0_hbm, e1_hbm, e2_hbm, g_hbm,
                   i0, i1, i2, oid, r0, r1, r2, acc,
                   gsem0, gsem1, wsem0, wsem1):
        wid = lax.axis_index("s") * NC + lax.axis_index("c")
        tbase = wid * TOK_PER_W
        pltpu.sync_copy(ct_hbm.at[0, pl.ds(tbase, TOK_PER_W)], i0)
        pltpu.sync_copy(ct_hbm.at[1, pl.ds(tbase, TOK_PER_W)], i1)
        pltpu.sync_copy(ct_hbm.at[2, pl.ds(tbase, TOK_PER_W)], i2)
        pltpu.sync_copy(oidx_hbm.at[pl.ds(wid * N_CHUNK, N_CHUNK)], oid)

        gsems = (gsem0, gsem1)
        wsems = (wsem0, wsem1)

        def issue(j, p):
            off = j * CHUNK
            pltpu.async_copy(e0_hbm.at[i0.at[pl.ds(off, CHUNK)]], r0.at[p], gsems[p])
            pltpu.async_copy(e1_hbm.at[i1.at[pl.ds(off, CHUNK)]], r1.at[p], gsems[p])
            pltpu.async_copy(e2_hbm.at[i2.at[pl.ds(off, CHUNK)]], r2.at[p], gsems[p])

        def wait_gather(j, p):
            off = j * CHUNK
            pltpu.make_async_copy(
                e0_hbm.at[i0.at[pl.ds(off, CHUNK)]], r0.at[p], gsems[p]).wait()
            pltpu.make_async_copy(
                e1_hbm.at[i1.at[pl.ds(off, CHUNK)]], r1.at[p], gsems[p]).wait()
            pltpu.make_async_copy(
                e2_hbm.at[i2.at[pl.ds(off, CHUNK)]], r2.at[p], gsems[p]).wait()

        def compute(p):
            def body(tok, carry):
                for k in range(VPT):
                    sl = pl.ds(k * L, L)
                    acc[p, tok, sl] = r0[p, tok, sl] + r1[p, tok, sl] + r2[p, tok, sl]
                return carry
            lax.fori_loop(0, CHUNK, body, 0)

        def issue_write(j, p):
            pltpu.async_copy(acc.at[p], g_hbm.at[oid.at[j]], wsems[p])

        def wait_write(j, p):
            pltpu.make_async_copy(acc.at[p], g_hbm.at[oid.at[j]], wsems[p]).wait()

        issue(0, 0)

        def step(t, carry):
            issue(2 * t + 1, 1)
            wait_gather(2 * t, 0)

            @pl.when(t > 0)
            def _():
                wait_write(2 * t - 2, 0)

            compute(0)
            issue_write(2 * t, 0)

            @pl.when(t < N_CHUNK // 2 - 1)
            def _():
                issue(2 * t + 2, 0)

            wait_gather(2 * t + 1, 1)

            @pl.when(t > 0)
            def _():
                wait_write(2 * t - 1, 1)

            compute(1)
            issue_write(2 * t + 1, 1)
            return carry

        lax.fori_loop(0, N_CHUNK // 2, step, 0)
        wait_write(N_CHUNK - 2, 0)
        wait_write(N_CHUNK - 1, 1)

    return gather_sum(ct, oidx2, e0, e1, e2)


BBT = 2048
NBB = B // BBT
HB = BBT // 2


def _tc_body(x_ref, w_ref, p3_ref, g_ref, o_ref):
    x = x_ref[0]
    yt = lax.dot_general(w_ref[...], x, (((0,), (1,)), ((), ())),
                         preferred_element_type=jnp.float32)
    gp = g_ref[...]
    ge_t = gp[:, :HIDDEN].T
    go_t = gp[:, HIDDEN:].T
    gt = jnp.concatenate([ge_t, go_t], axis=1)
    o_ref[0] = yt + p3_ref[0] + gt


def kernel(segments, semantic_embeds, categories, W, b, E0, E1, E2):
    del segments
    ct = categories.transpose(2, 0, 1).reshape(3, N_TOK)
    tok = jnp.arange(N_TOK, dtype=jnp.int32)
    bb = tok // S
    ss = tok % S
    orows = (2 * (ss * (NBB * HB) + (bb // BBT) * HB + bb % HB)
             + (bb % BBT) // HB)
    oidx2 = orows.reshape(N_TOK // CHUNK, CHUNK)

    g_t = _sc_gather_scatter_sum(ct, oidx2, E0, E1, E2)
    g2 = g_t.reshape(N2, 2 * HIDDEN)

    sem_t = jnp.transpose(semantic_embeds, (1, 0, 2))
    pe_b = jnp.asarray(_PE) + b[None, :]
    p3 = pe_b[:, :, None]

    out3 = pl.pallas_call(
        _tc_body,
        grid=(S, NBB),
        in_specs=[
            pl.BlockSpec((1, BBT, EMBED_LEN), lambda s, i: (s, i, 0)),
            pl.BlockSpec((EMBED_LEN, HIDDEN), lambda s, i: (0, 0)),
            pl.BlockSpec((1, HIDDEN, 1), lambda s, i: (s, 0, 0)),
            pl.BlockSpec((HB, 2 * HIDDEN), lambda s, i: (s * NBB + i, 0)),
        ],
        out_specs=pl.BlockSpec((1, HIDDEN, BBT), lambda s, i: (s, 0, i)),
        out_shape=jax.ShapeDtypeStruct((S, HIDDEN, B), jnp.float32),
    )(sem_t, W, p3, g2)
    return jnp.transpose(out3, (2, 0, 1))


# speedup vs baseline: 8.5347x; 1.2773x over previous
import functools
import math

import jax
import jax.numpy as jnp
import numpy as np
from jax import lax
from jax.experimental import pallas as pl
from jax.experimental.pallas import tpu as pltpu
from jax.experimental.pallas import tpu_sc as plsc

B, S, EMBED_LEN, HIDDEN = 4096, 50, 128, 64
MAX_LEN = 256

NC, NS, L = 2, 16, 16
NW = NC * NS
N_TOK = B * S
N2 = N_TOK // 2
TOK_PER_W = N_TOK // NW
CHUNK = 128
N_CHUNK = TOK_PER_W // CHUNK
VPT = HIDDEN // L


def _pe_np(d_model=HIDDEN, max_len=MAX_LEN):
    position = np.arange(0, max_len, dtype=np.float32)[:, None]
    div_term = np.exp(
        np.arange(0, d_model, 2, dtype=np.float32) * -(math.log(10000.0) / d_model))
    pe = np.zeros((max_len, d_model), dtype=np.float32)
    odd_len = d_model - div_term.shape[-1]
    pe[:, 0::2] = np.sin(position * div_term)
    pe[:, 1::2] = np.cos(position * div_term[:odd_len])
    return pe


_PE = _pe_np()[:S]


def _sc_gather_scatter_sum(ct, oidx2, e0, e1, e2):
    mesh = plsc.VectorSubcoreMesh(
        core_axis_name="c", subcore_axis_name="s",
        num_cores=NC, num_subcores=NS)

    @functools.partial(
        pl.kernel,
        out_type=jax.ShapeDtypeStruct((N_TOK, HIDDEN), jnp.float32),
        mesh=mesh,
        compiler_params=pltpu.CompilerParams(use_tc_tiling_on_sc=False),
        scratch_types=[
            pltpu.VMEM((TOK_PER_W,), jnp.int32),
            pltpu.VMEM((TOK_PER_W,), jnp.int32),
            pltpu.VMEM((TOK_PER_W,), jnp.int32),
            pltpu.VMEM((N_CHUNK, CHUNK), jnp.int32),
            pltpu.VMEM((2, CHUNK, HIDDEN), jnp.float32),
            pltpu.VMEM((2, CHUNK, HIDDEN), jnp.float32),
            pltpu.VMEM((2, CHUNK, HIDDEN), jnp.float32),
            pltpu.VMEM((2, CHUNK, HIDDEN), jnp.float32),
            pltpu.SemaphoreType.DMA,
            pltpu.SemaphoreType.DMA,
            pltpu.SemaphoreType.DMA,
            pltpu.SemaphoreType.DMA,
        ],
    )
    def gather_sum(ct_hbm, oidx_hbm, e0_hbm, e1_hbm, e2_hbm, g_hbm,
                   i0, i1, i2, oid, r0, r1, r2, acc,
                   gsem0, gsem1, wsem0, wsem1):
        wid = lax.axis_index("s") * NC + lax.axis_index("c")
        tbase = wid * TOK_PER_W
        pltpu.sync_copy(ct_hbm.at[0, pl.ds(tbase, TOK_PER_W)], i0)
        pltpu.sync_copy(ct_hbm.at[1, pl.ds(tbase, TOK_PER_W)], i1)
        pltpu.sync_copy(ct_hbm.at[2, pl.ds(tbase, TOK_PER_W)], i2)
        pltpu.sync_copy(oidx_hbm.at[pl.ds(wid * N_CHUNK, N_CHUNK)], oid)

        gsems = (gsem0, gsem1)
        wsems = (wsem0, wsem1)

        def issue(j, p):
            off = j * CHUNK
            pltpu.async_copy(e0_hbm.at[i0.at[pl.ds(off, CHUNK)]], r0.at[p], gsems[p])
            pltpu.async_copy(e1_hbm.at[i1.at[pl.ds(off, CHUNK)]], r1.at[p], gsems[p])
            pltpu.async_copy(e2_hbm.at[i2.at[pl.ds(off, CHUNK)]], r2.at[p], gsems[p])

        def wait_gather(j, p):
            off = j * CHUNK
            pltpu.make_async_copy(
                e0_hbm.at[i0.at[pl.ds(off, CHUNK)]], r0.at[p], gsems[p]).wait()
            pltpu.make_async_copy(
                e1_hbm.at[i1.at[pl.ds(off, CHUNK)]], r1.at[p], gsems[p]).wait()
            pltpu.make_async_copy(
                e2_hbm.at[i2.at[pl.ds(off, CHUNK)]], r2.at[p], gsems[p]).wait()

        def compute(p):
            def body(tok, carry):
                for k in range(VPT):
                    sl = pl.ds(k * L, L)
                    acc[p, tok, sl] = r0[p, tok, sl] + r1[p, tok, sl] + r2[p, tok, sl]
                return carry
            lax.fori_loop(0, CHUNK, body, 0)

        def issue_write(j, p):
            pltpu.async_copy(acc.at[p], g_hbm.at[oid.at[j]], wsems[p])

        def wait_write(j, p):
            pltpu.make_async_copy(acc.at[p], g_hbm.at[oid.at[j]], wsems[p]).wait()

        issue(0, 0)

        def step(t, carry):
            issue(2 * t + 1, 1)
            wait_gather(2 * t, 0)

            @pl.when(t > 0)
            def _():
                wait_write(2 * t - 2, 0)

            compute(0)
            issue_write(2 * t, 0)

            @pl.when(t < N_CHUNK // 2 - 1)
            def _():
                issue(2 * t + 2, 0)

            wait_gather(2 * t + 1, 1)

            @pl.when(t > 0)
            def _():
                wait_write(2 * t - 1, 1)

            compute(1)
            issue_write(2 * t + 1, 1)
            return carry

        lax.fori_loop(0, N_CHUNK // 2, step, 0)
        wait_write(N_CHUNK - 2, 0)
        wait_write(N_CHUNK - 1, 1)

    return gather_sum(ct, oidx2, e0, e1, e2)


HSPLIT = 50048
TBK = 2176
NBK = HSPLIT // TBK


def _tr_body(x1_ref, x2_ref, o_ref):
    o_ref[...] = jnp.concatenate([x1_ref[...].T, x2_ref[...].T], axis=1)


def _pack_table(e):
    et = e.T
    t2 = pl.pallas_call(
        _tr_body,
        grid=(NBK,),
        in_specs=[
            pl.BlockSpec((HIDDEN, TBK), lambda i: (0, i)),
            pl.BlockSpec((HIDDEN, TBK), lambda i: (0, i + NBK)),
        ],
        out_specs=pl.BlockSpec((TBK, 2 * HIDDEN), lambda i: (i, 0)),
        out_shape=jax.ShapeDtypeStruct((HSPLIT, 2 * HIDDEN), jnp.float32),
    )(et, et)
    return t2.reshape(2 * HSPLIT, HIDDEN)


BBT = 4096
NBB = B // BBT
HB = BBT // 2


def _tc_body(x_ref, w_ref, p3_ref, g_ref, o_ref):
    x = x_ref[0]
    yt = lax.dot_general(w_ref[...], x, (((0,), (1,)), ((), ())),
                         preferred_element_type=jnp.float32)
    gp = g_ref[...]
    ge_t = gp[:, :HIDDEN].T
    go_t = gp[:, HIDDEN:].T
    gt = jnp.concatenate([ge_t, go_t], axis=1)
    o_ref[0] = yt + p3_ref[0] + gt


def kernel(segments, semantic_embeds, categories, W, b, E0, E1, E2):
    del segments
    ct = categories.transpose(2, 0, 1).reshape(3, N_TOK)
    ct = jnp.where(ct < HSPLIT, 2 * ct, 2 * (ct - HSPLIT) + 1)
    tok = jnp.arange(N_TOK, dtype=jnp.int32)
    bb = tok // S
    ss = tok % S
    orows = (2 * (ss * (NBB * HB) + (bb // BBT) * HB + bb % HB)
             + (bb % BBT) // HB)
    oidx2 = orows.reshape(N_TOK // CHUNK, CHUNK)

    g_t = _sc_gather_scatter_sum(ct, oidx2, _pack_table(E0), _pack_table(E1),
                                 _pack_table(E2))
    g2 = g_t.reshape(N2, 2 * HIDDEN)

    sem_t = jnp.transpose(semantic_embeds, (1, 0, 2))
    pe_b = jnp.asarray(_PE) + b[None, :]
    p3 = pe_b[:, :, None]

    out3 = pl.pallas_call(
        _tc_body,
        grid=(S, NBB),
        in_specs=[
            pl.BlockSpec((1, BBT, EMBED_LEN), lambda s, i: (s, i, 0)),
            pl.BlockSpec((EMBED_LEN, HIDDEN), lambda s, i: (0, 0)),
            pl.BlockSpec((1, HIDDEN, 1), lambda s, i: (s, 0, 0)),
            pl.BlockSpec((HB, 2 * HIDDEN), lambda s, i: (s * NBB + i, 0)),
        ],
        out_specs=pl.BlockSpec((1, HIDDEN, BBT), lambda s, i: (s, 0, i)),
        out_shape=jax.ShapeDtypeStruct((S, HIDDEN, B), jnp.float32),
    )(sem_t, W, p3, g2)
    return jnp.transpose(out3, (2, 0, 1))


# speedup vs baseline: 8.7282x; 1.0227x over previous
import functools
import math

import jax
import jax.numpy as jnp
import numpy as np
from jax import lax
from jax.experimental import pallas as pl
from jax.experimental.pallas import tpu as pltpu
from jax.experimental.pallas import tpu_sc as plsc

B, S, EMBED_LEN, HIDDEN = 4096, 50, 128, 64
MAX_LEN = 256

NC, NS, L = 2, 16, 16
NW = NC * NS
N_TOK = B * S
N2 = N_TOK // 2
TOK_PER_W = N_TOK // NW
CHUNK = 128
N_CHUNK = TOK_PER_W // CHUNK
VPT = HIDDEN // L


def _pe_np(d_model=HIDDEN, max_len=MAX_LEN):
    position = np.arange(0, max_len, dtype=np.float32)[:, None]
    div_term = np.exp(
        np.arange(0, d_model, 2, dtype=np.float32) * -(math.log(10000.0) / d_model))
    pe = np.zeros((max_len, d_model), dtype=np.float32)
    odd_len = d_model - div_term.shape[-1]
    pe[:, 0::2] = np.sin(position * div_term)
    pe[:, 1::2] = np.cos(position * div_term[:odd_len])
    return pe


_PE = _pe_np()[:S]


def _sc_gather_scatter_sum(ct, oidx2, e0, e1, e2):
    mesh = plsc.VectorSubcoreMesh(
        core_axis_name="c", subcore_axis_name="s",
        num_cores=NC, num_subcores=NS)

    @functools.partial(
        pl.kernel,
        out_type=jax.ShapeDtypeStruct((N_TOK, HIDDEN), jnp.float32),
        mesh=mesh,
        compiler_params=pltpu.CompilerParams(use_tc_tiling_on_sc=False),
        scratch_types=[
            pltpu.VMEM((TOK_PER_W,), jnp.int32),
            pltpu.VMEM((TOK_PER_W,), jnp.int32),
            pltpu.VMEM((TOK_PER_W,), jnp.int32),
            pltpu.VMEM((N_CHUNK, CHUNK), jnp.int32),
            pltpu.VMEM((2, CHUNK, HIDDEN), jnp.float32),
            pltpu.VMEM((2, CHUNK, HIDDEN), jnp.float32),
            pltpu.VMEM((2, CHUNK, HIDDEN), jnp.float32),
            pltpu.VMEM((2, CHUNK, HIDDEN), jnp.float32),
            pltpu.SemaphoreType.DMA,
            pltpu.SemaphoreType.DMA,
            pltpu.SemaphoreType.DMA,
            pltpu.SemaphoreType.DMA,
        ],
    )
    def gather_sum(ct_hbm, oidx_hbm, e0_hbm, e1_hbm, e2_hbm, g_hbm,
                   i0, i1, i2, oid, r0, r1, r2, acc,
                   gsem0, gsem1, wsem0, wsem1):
        wid = lax.axis_index("s") * NC + lax.axis_index("c")
        tbase = wid * TOK_PER_W
        pltpu.sync_copy(ct_hbm.at[0, pl.ds(tbase, TOK_PER_W)], i0)
        pltpu.sync_copy(ct_hbm.at[1, pl.ds(tbase, TOK_PER_W)], i1)
        pltpu.sync_copy(ct_hbm.at[2, pl.ds(tbase, TOK_PER_W)], i2)
        pltpu.sync_copy(oidx_hbm.at[pl.ds(wid * N_CHUNK, N_CHUNK)], oid)

        gsems = (gsem0, gsem1)
        wsems = (wsem0, wsem1)

        def issue(j, p):
            off = j * CHUNK
            pltpu.async_copy(e0_hbm.at[i0.at[pl.ds(off, CHUNK)]], r0.at[p], gsems[p])
            pltpu.async_copy(e1_hbm.at[i1.at[pl.ds(off, CHUNK)]], r1.at[p], gsems[p])
            pltpu.async_copy(e2_hbm.at[i2.at[pl.ds(off, CHUNK)]], r2.at[p], gsems[p])

        def wait_gather(j, p):
            off = j * CHUNK
            pltpu.make_async_copy(
                e0_hbm.at[i0.at[pl.ds(off, CHUNK)]], r0.at[p], gsems[p]).wait()
            pltpu.make_async_copy(
                e1_hbm.at[i1.at[pl.ds(off, CHUNK)]], r1.at[p], gsems[p]).wait()
            pltpu.make_async_copy(
                e2_hbm.at[i2.at[pl.ds(off, CHUNK)]], r2.at[p], gsems[p]).wait()

        def compute(p):
            def body(tok, carry):
                for k in range(VPT):
                    sl = pl.ds(k * L, L)
                    acc[p, tok, sl] = r0[p, tok, sl] + r1[p, tok, sl] + r2[p, tok, sl]
                return carry
            lax.fori_loop(0, CHUNK, body, 0)

        def issue_write(j, p):
            pltpu.async_copy(acc.at[p], g_hbm.at[oid.at[j]], wsems[p])

        def wait_write(j, p):
            pltpu.make_async_copy(acc.at[p], g_hbm.at[oid.at[j]], wsems[p]).wait()

        issue(0, 0)

        def step(t, carry):
            issue(2 * t + 1, 1)
            wait_gather(2 * t, 0)

            @pl.when(t > 0)
            def _():
                wait_write(2 * t - 2, 0)

            compute(0)
            issue_write(2 * t, 0)

            @pl.when(t < N_CHUNK // 2 - 1)
            def _():
                issue(2 * t + 2, 0)

            wait_gather(2 * t + 1, 1)

            @pl.when(t > 0)
            def _():
                wait_write(2 * t - 1, 1)

            compute(1)
            issue_write(2 * t + 1, 1)
            return carry

        lax.fori_loop(0, N_CHUNK // 2, step, 0)
        wait_write(N_CHUNK - 2, 0)
        wait_write(N_CHUNK - 1, 1)

    return gather_sum(ct, oidx2, e0, e1, e2)


HSPLIT = 50048
TBK = 2176
NBK = HSPLIT // TBK


def _tr_body(x1_ref, x2_ref, eye_ref, o_ref):
    dn = (((0,), (0,)), ((), ()))
    e = eye_ref[...]
    o_ref[...] = jnp.concatenate(
        [lax.dot_general(x1_ref[...], e, dn, preferred_element_type=jnp.float32),
         lax.dot_general(x2_ref[...], e, dn, preferred_element_type=jnp.float32)],
        axis=1)


def _pack_table(e):
    et = e.T
    eye = jnp.eye(HIDDEN, dtype=jnp.float32)
    t2 = pl.pallas_call(
        _tr_body,
        grid=(NBK,),
        in_specs=[
            pl.BlockSpec((HIDDEN, TBK), lambda i: (0, i)),
            pl.BlockSpec((HIDDEN, TBK), lambda i: (0, i + NBK)),
            pl.BlockSpec((HIDDEN, HIDDEN), lambda i: (0, 0)),
        ],
        out_specs=pl.BlockSpec((TBK, 2 * HIDDEN), lambda i: (i, 0)),
        out_shape=jax.ShapeDtypeStruct((HSPLIT, 2 * HIDDEN), jnp.float32),
    )(et, et, eye)
    return t2.reshape(2 * HSPLIT, HIDDEN)


BBT = 4096
NBB = B // BBT
HB = BBT // 2


def _tc_body(x_ref, w_ref, p3_ref, g_ref, eye_ref, o_ref):
    x = x_ref[0]
    yt = lax.dot_general(w_ref[...], x, (((0,), (1,)), ((), ())),
                         preferred_element_type=jnp.float32)
    gp = g_ref[...]
    e = eye_ref[...]
    dn = (((1,), (1,)), ((), ()))
    ge_t = lax.dot_general(e, gp[:, :HIDDEN], dn,
                           preferred_element_type=jnp.float32)
    go_t = lax.dot_general(e, gp[:, HIDDEN:], dn,
                           preferred_element_type=jnp.float32)
    gt = jnp.concatenate([ge_t, go_t], axis=1)
    o_ref[0] = yt + p3_ref[0] + gt


def kernel(segments, semantic_embeds, categories, W, b, E0, E1, E2):
    del segments
    ct = categories.transpose(2, 0, 1).reshape(3, N_TOK)
    ct = jnp.where(ct < HSPLIT, 2 * ct, 2 * (ct - HSPLIT) + 1)
    tok = jnp.arange(N_TOK, dtype=jnp.int32)
    bb = tok // S
    ss = tok % S
    orows = (2 * (ss * (NBB * HB) + (bb // BBT) * HB + bb % HB)
             + (bb % BBT) // HB)
    oidx2 = orows.reshape(N_TOK // CHUNK, CHUNK)

    g_t = _sc_gather_scatter_sum(ct, oidx2, _pack_table(E0), _pack_table(E1),
                                 _pack_table(E2))
    g2 = g_t.reshape(N2, 2 * HIDDEN)

    sem_t = jnp.transpose(semantic_embeds, (1, 0, 2))
    pe_b = jnp.asarray(_PE) + b[None, :]
    p3 = pe_b[:, :, None]

    out3 = pl.pallas_call(
        _tc_body,
        grid=(S, NBB),
        in_specs=[
            pl.BlockSpec((1, BBT, EMBED_LEN), lambda s, i: (s, i, 0)),
            pl.BlockSpec((EMBED_LEN, HIDDEN), lambda s, i: (0, 0)),
            pl.BlockSpec((1, HIDDEN, 1), lambda s, i: (s, 0, 0)),
            pl.BlockSpec((HB, 2 * HIDDEN), lambda s, i: (s * NBB + i, 0)),
            pl.BlockSpec((HIDDEN, HIDDEN), lambda s, i: (0, 0)),
        ],
        out_specs=pl.BlockSpec((1, HIDDEN, BBT), lambda s, i: (s, 0, i)),
        out_shape=jax.ShapeDtypeStruct((S, HIDDEN, B), jnp.float32),
    )(sem_t, W, p3, g2, jnp.eye(HIDDEN, dtype=jnp.float32))
    return jnp.transpose(out3, (2, 0, 1))


# speedup vs baseline: 9.0031x; 1.0315x over previous
import functools
import math

import jax
import jax.numpy as jnp
import numpy as np
from jax import lax
from jax.experimental import pallas as pl
from jax.experimental.pallas import tpu as pltpu
from jax.experimental.pallas import tpu_sc as plsc

B, S, EMBED_LEN, HIDDEN = 4096, 50, 128, 64
MAX_LEN = 256

NC, NS, L = 2, 16, 16
NW = NC * NS
N_TOK = B * S
N2 = N_TOK // 2
TOK_PER_W = N_TOK // NW
CHUNK = 128
N_CHUNK = TOK_PER_W // CHUNK
VPT = HIDDEN // L


def _pe_np(d_model=HIDDEN, max_len=MAX_LEN):
    position = np.arange(0, max_len, dtype=np.float32)[:, None]
    div_term = np.exp(
        np.arange(0, d_model, 2, dtype=np.float32) * -(math.log(10000.0) / d_model))
    pe = np.zeros((max_len, d_model), dtype=np.float32)
    odd_len = d_model - div_term.shape[-1]
    pe[:, 0::2] = np.sin(position * div_term)
    pe[:, 1::2] = np.cos(position * div_term[:odd_len])
    return pe


_PE = _pe_np()[:S]


def _sc_gather_scatter_sum(ct, oidx2, e0, e1, e2):
    mesh = plsc.VectorSubcoreMesh(
        core_axis_name="c", subcore_axis_name="s",
        num_cores=NC, num_subcores=NS)

    @functools.partial(
        pl.kernel,
        out_type=jax.ShapeDtypeStruct((N_TOK, HIDDEN), jnp.float32),
        mesh=mesh,
        compiler_params=pltpu.CompilerParams(use_tc_tiling_on_sc=False),
        scratch_types=[
            pltpu.VMEM((TOK_PER_W,), jnp.int32),
            pltpu.VMEM((TOK_PER_W,), jnp.int32),
            pltpu.VMEM((TOK_PER_W,), jnp.int32),
            pltpu.VMEM((N_CHUNK, CHUNK), jnp.int32),
            pltpu.VMEM((2, CHUNK, HIDDEN), jnp.float32),
            pltpu.VMEM((2, CHUNK, HIDDEN), jnp.float32),
            pltpu.VMEM((2, CHUNK, HIDDEN), jnp.float32),
            pltpu.VMEM((2, CHUNK, HIDDEN), jnp.float32),
            pltpu.SemaphoreType.DMA,
            pltpu.SemaphoreType.DMA,
            pltpu.SemaphoreType.DMA,
            pltpu.SemaphoreType.DMA,
        ],
    )
    def gather_sum(ct_hbm, oidx_hbm, e0_hbm, e1_hbm, e2_hbm, g_hbm,
                   i0, i1, i2, oid, r0, r1, r2, acc,
                   gsem0, gsem1, wsem0, wsem1):
        wid = lax.axis_index("s") * NC + lax.axis_index("c")
        tbase = wid * TOK_PER_W
        pltpu.sync_copy(ct_hbm.at[0, pl.ds(tbase, TOK_PER_W)], i0)
        pltpu.sync_copy(ct_hbm.at[1, pl.ds(tbase, TOK_PER_W)], i1)
        pltpu.sync_copy(ct_hbm.at[2, pl.ds(tbase, TOK_PER_W)], i2)
        pltpu.sync_copy(oidx_hbm.at[pl.ds(wid * N_CHUNK, N_CHUNK)], oid)

        gsems = (gsem0, gsem1)
        wsems = (wsem0, wsem1)

        def issue(j, p):
            off = j * CHUNK
            pltpu.async_copy(e0_hbm.at[i0.at[pl.ds(off, CHUNK)]], r0.at[p], gsems[p])
            pltpu.async_copy(e1_hbm.at[i1.at[pl.ds(off, CHUNK)]], r1.at[p], gsems[p])
            pltpu.async_copy(e2_hbm.at[i2.at[pl.ds(off, CHUNK)]], r2.at[p], gsems[p])

        def wait_gather(j, p):
            off = j * CHUNK
            pltpu.make_async_copy(
                e0_hbm.at[i0.at[pl.ds(off, CHUNK)]], r0.at[p], gsems[p]).wait()
            pltpu.make_async_copy(
                e1_hbm.at[i1.at[pl.ds(off, CHUNK)]], r1.at[p], gsems[p]).wait()
            pltpu.make_async_copy(
                e2_hbm.at[i2.at[pl.ds(off, CHUNK)]], r2.at[p], gsems[p]).wait()

        def compute(p):
            def body(tok, carry):
                for k in range(VPT):
                    sl = pl.ds(k * L, L)
                    acc[p, tok, sl] = r0[p, tok, sl] + r1[p, tok, sl] + r2[p, tok, sl]
                return carry
            lax.fori_loop(0, CHUNK, body, 0)

        def issue_write(j, p):
            pltpu.async_copy(acc.at[p], g_hbm.at[oid.at[j]], wsems[p])

        def wait_write(j, p):
            pltpu.make_async_copy(acc.at[p], g_hbm.at[oid.at[j]], wsems[p]).wait()

        issue(0, 0)

        def step(t, carry):
            issue(2 * t + 1, 1)
            wait_gather(2 * t, 0)

            @pl.when(t > 0)
            def _():
                wait_write(2 * t - 2, 0)

            compute(0)
            issue_write(2 * t, 0)

            @pl.when(t < N_CHUNK // 2 - 1)
            def _():
                issue(2 * t + 2, 0)

            wait_gather(2 * t + 1, 1)

            @pl.when(t > 0)
            def _():
                wait_write(2 * t - 1, 1)

            compute(1)
            issue_write(2 * t + 1, 1)
            return carry

        lax.fori_loop(0, N_CHUNK // 2, step, 0)
        wait_write(N_CHUNK - 2, 0)
        wait_write(N_CHUNK - 1, 1)

    return gather_sum(ct, oidx2, e0, e1, e2)


HSPLIT = 50048
TBK = 2944
NBK = HSPLIT // TBK


def _tr_body(x1_ref, x2_ref, eye_ref, o_ref):
    dn = (((0,), (0,)), ((), ()))
    e = eye_ref[...]
    o_ref[...] = jnp.concatenate(
        [lax.dot_general(x1_ref[...], e, dn, preferred_element_type=jnp.float32),
         lax.dot_general(x2_ref[...], e, dn, preferred_element_type=jnp.float32)],
        axis=1)


def _pack_table(e):
    et = e.T
    eye = jnp.eye(HIDDEN, dtype=jnp.float32)
    t2 = pl.pallas_call(
        _tr_body,
        grid=(NBK,),
        in_specs=[
            pl.BlockSpec((HIDDEN, TBK), lambda i: (0, i)),
            pl.BlockSpec((HIDDEN, TBK), lambda i: (0, i + NBK)),
            pl.BlockSpec((HIDDEN, HIDDEN), lambda i: (0, 0)),
        ],
        out_specs=pl.BlockSpec((TBK, 2 * HIDDEN), lambda i: (i, 0)),
        out_shape=jax.ShapeDtypeStruct((HSPLIT, 2 * HIDDEN), jnp.float32),
    )(et, et, eye)
    return t2.reshape(2 * HSPLIT, HIDDEN)


BBT = 4096
NBB = B // BBT
HB = BBT // 2


def _tc_body(x_ref, w_ref, p3_ref, g_ref, eye_ref, o_ref):
    x = x_ref[0]
    yt = lax.dot_general(w_ref[...], x, (((0,), (1,)), ((), ())),
                         preferred_element_type=jnp.float32)
    gp = g_ref[...]
    e = eye_ref[...]
    dn = (((1,), (1,)), ((), ()))
    ge_t = lax.dot_general(e, gp[:, :HIDDEN], dn,
                           preferred_element_type=jnp.float32)
    go_t = lax.dot_general(e, gp[:, HIDDEN:], dn,
                           preferred_element_type=jnp.float32)
    gt = jnp.concatenate([ge_t, go_t], axis=1)
    o_ref[0] = yt + p3_ref[0] + gt


def kernel(segments, semantic_embeds, categories, W, b, E0, E1, E2):
    del segments
    ct = categories.transpose(2, 0, 1).reshape(3, N_TOK)
    ct = jnp.where(ct < HSPLIT, 2 * ct, 2 * (ct - HSPLIT) + 1)
    tok = jnp.arange(N_TOK, dtype=jnp.int32)
    bb = tok // S
    ss = tok % S
    orows = (2 * (ss * (NBB * HB) + (bb // BBT) * HB + bb % HB)
             + (bb % BBT) // HB)
    oidx2 = orows.reshape(N_TOK // CHUNK, CHUNK)

    g_t = _sc_gather_scatter_sum(ct, oidx2, _pack_table(E0), _pack_table(E1),
                                 _pack_table(E2))
    g2 = g_t.reshape(N2, 2 * HIDDEN)

    sem_t = jnp.transpose(semantic_embeds, (1, 0, 2))
    pe_b = jnp.asarray(_PE) + b[None, :]
    p3 = pe_b[:, :, None]

    out3 = pl.pallas_call(
        _tc_body,
        grid=(S, NBB),
        in_specs=[
            pl.BlockSpec((1, BBT, EMBED_LEN), lambda s, i: (s, i, 0)),
            pl.BlockSpec((EMBED_LEN, HIDDEN), lambda s, i: (0, 0)),
            pl.BlockSpec((1, HIDDEN, 1), lambda s, i: (s, 0, 0)),
            pl.BlockSpec((HB, 2 * HIDDEN), lambda s, i: (s * NBB + i, 0)),
            pl.BlockSpec((HIDDEN, HIDDEN), lambda s, i: (0, 0)),
        ],
        out_specs=pl.BlockSpec((1, HIDDEN, BBT), lambda s, i: (s, 0, i)),
        out_shape=jax.ShapeDtypeStruct((S, HIDDEN, B), jnp.float32),
    )(sem_t, W, p3, g2, jnp.eye(HIDDEN, dtype=jnp.float32))
    return jnp.transpose(out3, (2, 0, 1))


# speedup vs baseline: 9.8052x; 1.0891x over previous
import functools
import math

import jax
import jax.numpy as jnp
import numpy as np
from jax import lax
from jax.experimental import pallas as pl
from jax.experimental.pallas import tpu as pltpu
from jax.experimental.pallas import tpu_sc as plsc

B, S, EMBED_LEN, HIDDEN = 4096, 50, 128, 64
MAX_LEN = 256

NC, NS, L = 2, 16, 16
NW = NC * NS
N_TOK = B * S
N2 = N_TOK // 2
TOK_PER_W = N_TOK // NW
CHUNK = 128
N_CHUNK = TOK_PER_W // CHUNK
VPT = HIDDEN // L


def _pe_np(d_model=HIDDEN, max_len=MAX_LEN):
    position = np.arange(0, max_len, dtype=np.float32)[:, None]
    div_term = np.exp(
        np.arange(0, d_model, 2, dtype=np.float32) * -(math.log(10000.0) / d_model))
    pe = np.zeros((max_len, d_model), dtype=np.float32)
    odd_len = d_model - div_term.shape[-1]
    pe[:, 0::2] = np.sin(position * div_term)
    pe[:, 1::2] = np.cos(position * div_term[:odd_len])
    return pe


_PE = _pe_np()[:S]


def _sc_gather_scatter_sum(ct, oidx2, e0, e1, e2):
    mesh = plsc.VectorSubcoreMesh(
        core_axis_name="c", subcore_axis_name="s",
        num_cores=NC, num_subcores=NS)

    @functools.partial(
        pl.kernel,
        out_type=jax.ShapeDtypeStruct((N_TOK, HIDDEN), jnp.float32),
        mesh=mesh,
        compiler_params=pltpu.CompilerParams(use_tc_tiling_on_sc=False),
        scratch_types=[
            pltpu.VMEM((TOK_PER_W,), jnp.int32),
            pltpu.VMEM((TOK_PER_W,), jnp.int32),
            pltpu.VMEM((TOK_PER_W,), jnp.int32),
            pltpu.VMEM((N_CHUNK, CHUNK), jnp.int32),
            pltpu.VMEM((2, CHUNK, HIDDEN), jnp.float32),
            pltpu.VMEM((2, CHUNK, HIDDEN), jnp.float32),
            pltpu.VMEM((2, CHUNK, HIDDEN), jnp.float32),
            pltpu.VMEM((2, CHUNK, HIDDEN), jnp.float32),
            pltpu.SemaphoreType.DMA,
            pltpu.SemaphoreType.DMA,
            pltpu.SemaphoreType.DMA,
            pltpu.SemaphoreType.DMA,
        ],
    )
    def gather_sum(ct_hbm, oidx_hbm, e0_hbm, e1_hbm, e2_hbm, g_hbm,
                   i0, i1, i2, oid, r0, r1, r2, acc,
                   gsem0, gsem1, wsem0, wsem1):
        wid = lax.axis_index("s") * NC + lax.axis_index("c")
        tbase = wid * TOK_PER_W
        pltpu.sync_copy(ct_hbm.at[0, pl.ds(tbase, TOK_PER_W)], i0)
        pltpu.sync_copy(ct_hbm.at[1, pl.ds(tbase, TOK_PER_W)], i1)
        pltpu.sync_copy(ct_hbm.at[2, pl.ds(tbase, TOK_PER_W)], i2)
        pltpu.sync_copy(oidx_hbm.at[pl.ds(wid * N_CHUNK, N_CHUNK)], oid)

        gsems = (gsem0, gsem1)
        wsems = (wsem0, wsem1)

        def issue(j, p):
            off = j * CHUNK
            pltpu.async_copy(e0_hbm.at[i0.at[pl.ds(off, CHUNK)]], r0.at[p], gsems[p])
            pltpu.async_copy(e1_hbm.at[i1.at[pl.ds(off, CHUNK)]], r1.at[p], gsems[p])
            pltpu.async_copy(e2_hbm.at[i2.at[pl.ds(off, CHUNK)]], r2.at[p], gsems[p])

        def wait_gather(j, p):
            off = j * CHUNK
            pltpu.make_async_copy(
                e0_hbm.at[i0.at[pl.ds(off, CHUNK)]], r0.at[p], gsems[p]).wait()
            pltpu.make_async_copy(
                e1_hbm.at[i1.at[pl.ds(off, CHUNK)]], r1.at[p], gsems[p]).wait()
            pltpu.make_async_copy(
                e2_hbm.at[i2.at[pl.ds(off, CHUNK)]], r2.at[p], gsems[p]).wait()

        def compute(p):
            def body(tok, carry):
                for k in range(VPT):
                    sl = pl.ds(k * L, L)
                    acc[p, tok, sl] = r0[p, tok, sl] + r1[p, tok, sl] + r2[p, tok, sl]
                return carry
            lax.fori_loop(0, CHUNK, body, 0)

        def issue_write(j, p):
            pltpu.async_copy(acc.at[p], g_hbm.at[oid.at[j]], wsems[p])

        def wait_write(j, p):
            pltpu.make_async_copy(acc.at[p], g_hbm.at[oid.at[j]], wsems[p]).wait()

        issue(0, 0)

        def step(t, carry):
            issue(2 * t + 1, 1)
            wait_gather(2 * t, 0)

            @pl.when(t > 0)
            def _():
                wait_write(2 * t - 2, 0)

            compute(0)
            issue_write(2 * t, 0)

            @pl.when(t < N_CHUNK // 2 - 1)
            def _():
                issue(2 * t + 2, 0)

            wait_gather(2 * t + 1, 1)

            @pl.when(t > 0)
            def _():
                wait_write(2 * t - 1, 1)

            compute(1)
            issue_write(2 * t + 1, 1)
            return carry

        lax.fori_loop(0, N_CHUNK // 2, step, 0)
        wait_write(N_CHUNK - 2, 0)
        wait_write(N_CHUNK - 1, 1)

    return gather_sum(ct, oidx2, e0, e1, e2)


HSPLIT = 50048
TBK = 2944
NBK = HSPLIT // TBK


def _tr_body(a1, a2, b1, b2, c1, c2, eye_ref, o0, o1, o2):
    dn = (((0,), (0,)), ((), ()))
    e = eye_ref[...]
    for x1, x2, o in ((a1, a2, o0), (b1, b2, o1), (c1, c2, o2)):
        o[...] = jnp.concatenate(
            [lax.dot_general(x1[...], e, dn, preferred_element_type=jnp.float32),
             lax.dot_general(x2[...], e, dn, preferred_element_type=jnp.float32)],
            axis=1)


def _pack_tables(e0, e1, e2):
    eye = jnp.eye(HIDDEN, dtype=jnp.float32)
    ins, specs = [], []
    for e in (e0, e1, e2):
        et = e.T
        ins += [et, et]
        specs += [pl.BlockSpec((HIDDEN, TBK), lambda i: (0, i)),
                  pl.BlockSpec((HIDDEN, TBK), lambda i: (0, i + NBK))]
    specs.append(pl.BlockSpec((HIDDEN, HIDDEN), lambda i: (0, 0)))
    outs = pl.pallas_call(
        _tr_body,
        grid=(NBK,),
        in_specs=specs,
        out_specs=[pl.BlockSpec((TBK, 2 * HIDDEN), lambda i: (i, 0))] * 3,
        out_shape=[jax.ShapeDtypeStruct((HSPLIT, 2 * HIDDEN), jnp.float32)] * 3,
    )(*ins, eye)
    return [t.reshape(2 * HSPLIT, HIDDEN) for t in outs]


BBT = 4096
NBB = B // BBT
HB = BBT // 2


def _tc_body(x_ref, w_ref, p3_ref, g_ref, eye_ref, o_ref):
    x = x_ref[0]
    yt = lax.dot_general(w_ref[...], x, (((0,), (1,)), ((), ())),
                         preferred_element_type=jnp.float32)
    gp = g_ref[...]
    e = eye_ref[...]
    dn = (((1,), (1,)), ((), ()))
    ge_t = lax.dot_general(e, gp[:, :HIDDEN], dn,
                           preferred_element_type=jnp.float32)
    go_t = lax.dot_general(e, gp[:, HIDDEN:], dn,
                           preferred_element_type=jnp.float32)
    gt = jnp.concatenate([ge_t, go_t], axis=1)
    o_ref[0] = yt + p3_ref[0] + gt


def kernel(segments, semantic_embeds, categories, W, b, E0, E1, E2):
    del segments
    ct = categories.transpose(2, 0, 1).reshape(3, N_TOK)
    ct = jnp.where(ct < HSPLIT, 2 * ct, 2 * (ct - HSPLIT) + 1)
    tok = jnp.arange(N_TOK, dtype=jnp.int32)
    bb = tok // S
    ss = tok % S
    orows = (2 * (ss * (NBB * HB) + (bb // BBT) * HB + bb % HB)
             + (bb % BBT) // HB)
    oidx2 = orows.reshape(N_TOK // CHUNK, CHUNK)

    t0, t1, t2 = _pack_tables(E0, E1, E2)
    g_t = _sc_gather_scatter_sum(ct, oidx2, t0, t1, t2)
    g2 = g_t.reshape(N2, 2 * HIDDEN)

    sem_t = jnp.transpose(semantic_embeds, (1, 0, 2))
    pe_b = jnp.asarray(_PE) + b[None, :]
    p3 = pe_b[:, :, None]

    out3 = pl.pallas_call(
        _tc_body,
        grid=(S, NBB),
        in_specs=[
            pl.BlockSpec((1, BBT, EMBED_LEN), lambda s, i: (s, i, 0)),
            pl.BlockSpec((EMBED_LEN, HIDDEN), lambda s, i: (0, 0)),
            pl.BlockSpec((1, HIDDEN, 1), lambda s, i: (s, 0, 0)),
            pl.BlockSpec((HB, 2 * HIDDEN), lambda s, i: (s * NBB + i, 0)),
            pl.BlockSpec((HIDDEN, HIDDEN), lambda s, i: (0, 0)),
        ],
        out_specs=pl.BlockSpec((1, HIDDEN, BBT), lambda s, i: (s, 0, i)),
        out_shape=jax.ShapeDtypeStruct((S, HIDDEN, B), jnp.float32),
    )(sem_t, W, p3, g2, jnp.eye(HIDDEN, dtype=jnp.float32))
    return jnp.transpose(out3, (2, 0, 1))
